# baseline (speedup 1.0000x reference)
import jax
import jax.numpy as jnp
from jax.experimental import pallas as pl

_BLOCK = 2048
_SUB = 8


def _assign_block(x_ref, ct_ref, csq_ref, xsq_ref, assigns_ref, mins_ref):
    xb = x_ref[:]
    ct = ct_ref[:]
    csq = csq_ref[:]
    prod = jax.lax.dot_general(
        ct, xb, (((1,), (1,)), ((), ())),
        preferred_element_type=jnp.float32)
    dt = prod + csq
    k = dt.shape[0]
    ntiles = k // _SUB
    best = dt[0:_SUB, :]
    bidx = jnp.zeros(best.shape, jnp.int32)
    for t in range(1, ntiles):
        cur = dt[t * _SUB:(t + 1) * _SUB, :]
        pred = cur < best
        bidx = jnp.where(pred, t, bidx)
        best = jnp.minimum(best, cur)
    m = jnp.min(best, axis=0, keepdims=True)
    sub = jax.lax.broadcasted_iota(jnp.int32, best.shape, 0)
    gidx = bidx * _SUB + sub
    idx = jnp.min(jnp.where(best <= m, gidx, k), axis=0)
    assigns_ref[0, 0, :] = idx
    mins_ref[0, 0, :] = xsq_ref[0, :] + m[0, :]


def kernel(x, centroids):
    n, dim = x.shape
    k = centroids.shape[0]
    cm2 = -2.0 * centroids
    c_sq = jnp.sum(centroids * centroids, axis=1, keepdims=True)
    x_sq = jnp.sum(x * x, axis=1)[None, :]
    grid = (n // _BLOCK,)
    assigns, mins = pl.pallas_call(
        _assign_block,
        grid=grid,
        in_specs=[
            pl.BlockSpec((_BLOCK, dim), lambda i: (i, 0)),
            pl.BlockSpec((k, dim), lambda i: (0, 0)),
            pl.BlockSpec((k, 1), lambda i: (0, 0)),
            pl.BlockSpec((1, _BLOCK), lambda i: (0, i)),
        ],
        out_specs=[
            pl.BlockSpec((1, 1, _BLOCK), lambda i: (i, 0, 0)),
            pl.BlockSpec((1, 1, _BLOCK), lambda i: (i, 0, 0)),
        ],
        out_shape=[
            jax.ShapeDtypeStruct((n // _BLOCK, 1, _BLOCK), jnp.int32),
            jax.ShapeDtypeStruct((n // _BLOCK, 1, _BLOCK), jnp.float32),
        ],
    )(x, cm2, c_sq, x_sq)
    return assigns.reshape(n), mins.reshape(n)


# speedup vs baseline: 1.9104x; 1.9104x over previous
import jax
import jax.numpy as jnp
from jax.experimental import pallas as pl

_BLOCK = 2048
_SUB = 8


def _assign_block(xt_ref, ct_ref, csq_ref, assigns_ref, mins_ref):
    xt = xt_ref[:]
    ct = ct_ref[:]
    csq = csq_ref[:]
    dt = jnp.dot(ct, xt, preferred_element_type=jnp.float32) + csq
    k = dt.shape[0]
    ntiles = k // _SUB
    best = dt[0:_SUB, :]
    bidx = jnp.zeros(best.shape, jnp.int32)
    for t in range(1, ntiles):
        cur = dt[t * _SUB:(t + 1) * _SUB, :]
        pred = cur < best
        bidx = jnp.where(pred, t, bidx)
        best = jnp.minimum(best, cur)
    m = jnp.min(best, axis=0, keepdims=True)
    sub = jax.lax.broadcasted_iota(jnp.int32, best.shape, 0)
    gidx = bidx * _SUB + sub
    idx = jnp.min(jnp.where(best <= m, gidx, k), axis=0)
    x_sq = jnp.sum(xt * xt, axis=0, keepdims=True)
    assigns_ref[0, 0, :] = idx
    mins_ref[0, 0, :] = x_sq[0, :] + m[0, :]


def kernel(x, centroids):
    n, dim = x.shape
    k = centroids.shape[0]
    cm2 = -2.0 * centroids
    c_sq = jnp.sum(centroids * centroids, axis=1, keepdims=True)
    xt = x.T
    grid = (n // _BLOCK,)
    assigns, mins = pl.pallas_call(
        _assign_block,
        grid=grid,
        in_specs=[
            pl.BlockSpec((dim, _BLOCK), lambda i: (0, i)),
            pl.BlockSpec((k, dim), lambda i: (0, 0)),
            pl.BlockSpec((k, 1), lambda i: (0, 0)),
        ],
        out_specs=[
            pl.BlockSpec((1, 1, _BLOCK), lambda i: (i, 0, 0)),
            pl.BlockSpec((1, 1, _BLOCK), lambda i: (i, 0, 0)),
        ],
        out_shape=[
            jax.ShapeDtypeStruct((n // _BLOCK, 1, _BLOCK), jnp.int32),
            jax.ShapeDtypeStruct((n // _BLOCK, 1, _BLOCK), jnp.float32),
        ],
    )(xt, cm2, c_sq)
    return assigns.reshape(n), mins.reshape(n)


# speedup vs baseline: 2.3057x; 1.2069x over previous
import jax
import jax.numpy as jnp
from jax.experimental import pallas as pl

_BLOCK = 4096
_SUB = 8


def _assign_block(xt_ref, ct_ref, csq_ref, assigns_ref, mins_ref):
    xt = xt_ref[:]
    ct = ct_ref[:]
    csq = csq_ref[:]
    dt = jnp.dot(ct, xt, preferred_element_type=jnp.float32) + csq
    k = dt.shape[0]
    ntiles = k // _SUB
    best = dt[0:_SUB, :]
    bidx = jnp.zeros(best.shape, jnp.int32)
    for t in range(1, ntiles):
        cur = dt[t * _SUB:(t + 1) * _SUB, :]
        pred = cur < best
        bidx = jnp.where(pred, t, bidx)
        best = jnp.minimum(best, cur)
    m = jnp.min(best, axis=0, keepdims=True)
    sub = jax.lax.broadcasted_iota(jnp.int32, best.shape, 0)
    gidx = bidx * _SUB + sub
    idx = jnp.min(jnp.where(best <= m, gidx, k), axis=0)
    x_sq = jnp.sum(xt * xt, axis=0, keepdims=True)
    assigns_ref[0, 0, :] = idx
    mins_ref[0, 0, :] = x_sq[0, :] + m[0, :]


def kernel(x, centroids):
    n, dim = x.shape
    k = centroids.shape[0]
    cm2 = -2.0 * centroids
    c_sq = jnp.sum(centroids * centroids, axis=1, keepdims=True)
    xt = x.T
    grid = (n // _BLOCK,)
    assigns, mins = pl.pallas_call(
        _assign_block,
        grid=grid,
        in_specs=[
            pl.BlockSpec((dim, _BLOCK), lambda i: (0, i)),
            pl.BlockSpec((k, dim), lambda i: (0, 0)),
            pl.BlockSpec((k, 1), lambda i: (0, 0)),
        ],
        out_specs=[
            pl.BlockSpec((1, 1, _BLOCK), lambda i: (i, 0, 0)),
            pl.BlockSpec((1, 1, _BLOCK), lambda i: (i, 0, 0)),
        ],
        out_shape=[
            jax.ShapeDtypeStruct((n // _BLOCK, 1, _BLOCK), jnp.int32),
            jax.ShapeDtypeStruct((n // _BLOCK, 1, _BLOCK), jnp.float32),
        ],
    )(xt, cm2, c_sq)
    return assigns.reshape(n), mins.reshape(n)


# speedup vs baseline: 2.3201x; 1.0062x over previous
import jax
import jax.numpy as jnp
from jax.experimental import pallas as pl

_BLOCK = 8192
_SUB = 8


def _assign_block(xt_ref, ct_ref, csq_ref, assigns_ref, mins_ref):
    xt = xt_ref[:]
    ct = ct_ref[:]
    csq = csq_ref[:]
    dt = jnp.dot(ct, xt, preferred_element_type=jnp.float32) + csq
    k = dt.shape[0]
    ntiles = k // _SUB
    best = dt[0:_SUB, :]
    bidx = jnp.zeros(best.shape, jnp.int32)
    for t in range(1, ntiles):
        cur = dt[t * _SUB:(t + 1) * _SUB, :]
        pred = cur < best
        bidx = jnp.where(pred, t, bidx)
        best = jnp.minimum(best, cur)
    m = jnp.min(best, axis=0, keepdims=True)
    sub = jax.lax.broadcasted_iota(jnp.int32, best.shape, 0)
    gidx = bidx * _SUB + sub
    idx = jnp.min(jnp.where(best <= m, gidx, k), axis=0)
    x_sq = jnp.sum(xt * xt, axis=0, keepdims=True)
    assigns_ref[0, 0, :] = idx
    mins_ref[0, 0, :] = x_sq[0, :] + m[0, :]


def kernel(x, centroids):
    n, dim = x.shape
    k = centroids.shape[0]
    cm2 = -2.0 * centroids
    c_sq = jnp.sum(centroids * centroids, axis=1, keepdims=True)
    xt = x.T
    grid = (n // _BLOCK,)
    assigns, mins = pl.pallas_call(
        _assign_block,
        grid=grid,
        in_specs=[
            pl.BlockSpec((dim, _BLOCK), lambda i: (0, i)),
            pl.BlockSpec((k, dim), lambda i: (0, 0)),
            pl.BlockSpec((k, 1), lambda i: (0, 0)),
        ],
        out_specs=[
            pl.BlockSpec((1, 1, _BLOCK), lambda i: (i, 0, 0)),
            pl.BlockSpec((1, 1, _BLOCK), lambda i: (i, 0, 0)),
        ],
        out_shape=[
            jax.ShapeDtypeStruct((n // _BLOCK, 1, _BLOCK), jnp.int32),
            jax.ShapeDtypeStruct((n // _BLOCK, 1, _BLOCK), jnp.float32),
        ],
    )(xt, cm2, c_sq)
    return assigns.reshape(n), mins.reshape(n)
